# baseline (speedup 1.0000x reference)
import functools

import jax
import jax.numpy as jnp
from jax import lax
from jax.experimental import pallas as pl
from jax.experimental.pallas import tpu as pltpu
from jax.experimental.pallas import tpu_sc as plsc

N = 50000
E = 800000
H = 64
NG = 64
NPAD = 50048
BN = NPAD // 16
IDX_W = 128
ROWS_PER_TILE = 400
EPAD = 16 * ROWS_PER_TILE * IDX_W
IDX_ROWS = EPAD // IDX_W
EB = 256
CPB = 4
NBLKE = 50
EPT = NBLKE * CPB * EB
RING = 2
F32 = jnp.float32


def _sc_body(with_deg, *refs):
    if with_deg:
        (xv, idxh, z2, z1, msum_out, deg_out, acc, dega,
         istg0, istg1, rb0, rb1, ones_v, stage,
         sg0, sg1, ss0, ss1, sem_i, sem_d) = refs
    else:
        (xv, idxh, z2, msum_out, acc,
         istg0, istg1, rb0, rb1,
         sg0, sg1, ss0, ss1, sem_i) = refs
    istg = (istg0, istg1)
    rows = (rb0, rb1)
    sem_g = (sg0, sg1)
    sem_s = (ss0, ss1)
    c = lax.axis_index("c")
    s = lax.axis_index("s")
    r0 = s * BN
    pltpu.sync_copy(z2.at[pl.ds(r0, BN)], acc.at[pl.ds(r0, BN)])
    if with_deg:
        pltpu.sync_copy(z1.at[pl.ds(r0, BN)], stage)
        pltpu.sync_copy(stage, dega.at[pl.ds(r0, BN)])
        for j in range(EB // 16):
            ones_v[pl.ds(j * 16, 16)] = jnp.full((16,), 1.0, F32)
    plsc.subcore_barrier()

    ibase = (c * 16 + s) * NBLKE * 2 * CPB
    pltpu.sync_copy(idxh.at[pl.ds(ibase, 2 * CPB)], istg[0])

    def block(b, q):
        ss = istg[q]
        pre = pltpu.async_copy(
            idxh.at[pl.ds(ibase + (b + 1) * 2 * CPB, 2 * CPB)],
            istg[1 - q], sem_i)

        def issue_g(i):
            return pltpu.async_copy(xv.at[ss.at[2 * i]], rows[i % RING],
                                    sem_g[i % RING])

        def issue_s(i):
            return pltpu.async_copy(rows[i % RING], acc.at[ss.at[2 * i + 1]],
                                    sem_s[i % RING], add=True)

        g = {}
        sc = {}
        g[0] = issue_g(0)
        for i in range(CPB):
            g[i].wait()
            sc[i] = issue_s(i)
            if with_deg:
                @pl.when(c == i % 2)
                def _():
                    pltpu.async_copy(ones_v, dega.at[ss.at[2 * i + 1]],
                                     sem_d, add=True)
            if i + 1 < CPB:
                if i - 1 >= 0:
                    sc[i - 1].wait()
                g[i + 1] = issue_g(i + 1)
        if with_deg:
            for i in range(CPB):
                @pl.when(c == i % 2)
                def _():
                    pltpu.make_async_copy(
                        ones_v, dega.at[ss.at[2 * i + 1]], sem_d).wait()
        for t in range(max(0, CPB - RING), CPB):
            sc[t].wait()
        pre.wait()

    def two_blocks(j, carry):
        block(2 * j, 0)
        block(2 * j + 1, 1)
        return carry

    lax.fori_loop(0, NBLKE // 2, two_blocks, 0)
    plsc.subcore_barrier()
    pltpu.sync_copy(acc.at[pl.ds(r0, BN)],
                    msum_out.at[pl.ds(c * NPAD + r0, BN)])
    if with_deg:
        pltpu.sync_copy(dega.at[pl.ds(r0, BN)], stage)
        pltpu.sync_copy(stage, deg_out.at[pl.ds(c * NPAD + r0, BN)])


def _make_sc_pass(with_deg):
    mesh = plsc.VectorSubcoreMesh(core_axis_name="c", subcore_axis_name="s")
    out_type = (jax.ShapeDtypeStruct((2 * NPAD, 32), F32),)
    if with_deg:
        out_type = out_type + (jax.ShapeDtypeStruct((2 * NPAD,), F32),)
    scratch = [pltpu.VMEM_SHARED((NPAD, 32), F32)]
    if with_deg:
        scratch.append(pltpu.VMEM_SHARED((NPAD,), F32))
    scratch += [
        pltpu.VMEM((2 * CPB, EB), jnp.int32),
        pltpu.VMEM((2 * CPB, EB), jnp.int32),
        pltpu.VMEM((EB, 32), F32),
        pltpu.VMEM((EB, 32), F32),
    ]
    if with_deg:
        scratch.append(pltpu.VMEM((EB,), F32))
        scratch.append(pltpu.VMEM((BN,), F32))
    scratch += [pltpu.SemaphoreType.DMA] * (2 * RING + 1)
    if with_deg:
        scratch.append(pltpu.SemaphoreType.DMA)
    return pl.kernel(
        functools.partial(_sc_body, with_deg),
        out_type=out_type,
        mesh=mesh,
        scratch_types=scratch,
        compiler_params=pltpu.CompilerParams(use_tc_tiling_on_sc=False),
    )


def _pre_body(nf_ref, w_ref, b_ref, o_ref):
    o_ref[...] = (jnp.dot(nf_ref[...], w_ref[...],
                          preferred_element_type=F32) + b_ref[...])


def _layer0_body(x_ref, ma_ref, mb_ref, deg_ref, skip_ref, wl_ref, wr_ref,
                 b_ref, o_ref):
    s00 = jax.nn.sigmoid(skip_ref[0, 0])
    d = jnp.maximum(deg_ref[...], 1.0)
    m = jnp.concatenate([ma_ref[...], mb_ref[...]], axis=1) / d
    h = s00 * (jnp.dot(m, wl_ref[...], preferred_element_type=F32)
               + jnp.dot(x_ref[...], wr_ref[...], preferred_element_type=F32))
    o_ref[...] = jnp.maximum(h + b_ref[...], 0.0)


def _layer1_pool_body(x_ref, x1_ref, m0a_ref, m0b_ref, m1a_ref, m1b_ref,
                      deg_ref, batch_ref, skip_ref, wl_ref, wr_ref, b_ref,
                      pooled_ref):
    i = pl.program_id(0)
    s10 = jax.nn.sigmoid(skip_ref[1, 0])
    s11 = jax.nn.sigmoid(skip_ref[1, 1])
    d = jnp.maximum(deg_ref[...], 1.0)
    m0 = jnp.concatenate([m0a_ref[...], m0b_ref[...]], axis=1) / d
    m1 = jnp.concatenate([m1a_ref[...], m1b_ref[...]], axis=1) / d
    x = x_ref[...]
    x1 = x1_ref[...]
    wl = wl_ref[...]
    wr = wr_ref[...]
    h = (s10 * (jnp.dot(m0, wl[:H], preferred_element_type=F32)
                + jnp.dot(x, wr[:H], preferred_element_type=F32))
         + s11 * (jnp.dot(m1, wl[H:], preferred_element_type=F32)
                  + jnp.dot(x1, wr[H:], preferred_element_type=F32))
         + b_ref[...])
    x2 = jnp.maximum(h, 0.0)
    emb = jnp.concatenate([x, x1, x2], axis=1)
    b = batch_ref[0, 0, :]
    oh = (b[:, None] == lax.broadcasted_iota(jnp.int32, (BN, NG), 1)
          ).astype(F32)
    part = lax.dot_general(oh, emb, (((0,), (0,)), ((), ())),
                           preferred_element_type=F32)

    @pl.when(i == 0)
    def _():
        pooled_ref[...] = part

    @pl.when(i > 0)
    def _():
        pooled_ref[...] += part


def _mlp_body(p_ref, w1_ref, b1_ref, w2_ref, b2_ref, w3_ref, b3_ref,
              w4_ref, b4_ref, o_ref):
    h = jnp.dot(p_ref[...], w1_ref[...], preferred_element_type=F32) + b1_ref[...]
    h = jnp.where(h >= 0, h, 0.1 * h)
    h = jnp.maximum(jnp.dot(h, w2_ref[...], preferred_element_type=F32)
                    + b2_ref[...], 0.0)
    h = jnp.maximum(jnp.dot(h, w3_ref[...], preferred_element_type=F32)
                    + b3_ref[...], 0.0)
    o_ref[...] = jnp.dot(h, w4_ref[...], preferred_element_type=F32) + b4_ref[...]


def _full(shape):
    return pl.BlockSpec(shape, lambda i: tuple(0 for _ in shape))


def kernel(node_feature, edge_index, batch, learnable_skip, W_pre, b_pre,
           W0l, b0, W0r, W1l, b1, W1r,
           Wp1, bp1, Wp2, bp2, Wp3, bp3, Wp4, bp4):
    src = edge_index[0]
    dst = edge_index[1]
    pad = EPAD - E
    srcp = jnp.concatenate([src, jnp.zeros((pad,), jnp.int32)])
    dstp = jnp.concatenate([dst, jnp.full((pad,), N, jnp.int32)])
    dst4 = dstp.reshape(16, NBLKE, CPB, 1, EB)
    idxh = jnp.concatenate(
        [jnp.stack([(2 * srcp + cc).reshape(16, NBLKE, CPB, 1, EB)
                    for cc in range(2)]),
         jnp.stack([dst4, dst4])], axis=4
    ).reshape(2 * 16 * NBLKE * 2 * CPB, EB)
    idxh = jnp.pad(idxh, ((0, 2 * CPB), (0, 0)))
    nf_pad = jnp.pad(node_feature, ((0, NPAD - N), (0, 0)))
    batch3 = jnp.pad(batch, (0, NPAD - N), constant_values=NG
                     ).reshape(16, 1, BN)
    z32 = jnp.zeros((NPAD, 32), F32)
    z1 = jnp.zeros((NPAD,), F32)

    x_pad = pl.pallas_call(
        _pre_body,
        grid=(16,),
        in_specs=[pl.BlockSpec((BN, 5), lambda i: (i, 0)),
                  _full((5, H)), _full((1, H))],
        out_specs=pl.BlockSpec((BN, H), lambda i: (i, 0)),
        out_shape=jax.ShapeDtypeStruct((NPAD, H), F32),
    )(nf_pad, W_pre, b_pre.reshape(1, H))

    xv = x_pad.reshape(2 * NPAD, 32)
    msum0, degp = _make_sc_pass(True)(xv, idxh, z32, z1)
    deg2 = (degp[:NPAD] + degp[NPAD:]).reshape(NPAD, 1)

    mspec_a = pl.BlockSpec((BN, 32), lambda i: (i, 0))
    mspec_b = pl.BlockSpec((BN, 32), lambda i: (i + 16, 0))
    dspec = pl.BlockSpec((BN, 1), lambda i: (i, 0))
    x1_pad = pl.pallas_call(
        _layer0_body,
        grid=(16,),
        in_specs=[pl.BlockSpec((BN, H), lambda i: (i, 0)),
                  mspec_a, mspec_b, dspec,
                  _full((2, 2)), _full((H, H)), _full((H, H)), _full((1, H))],
        out_specs=pl.BlockSpec((BN, H), lambda i: (i, 0)),
        out_shape=jax.ShapeDtypeStruct((NPAD, H), F32),
    )(x_pad, msum0, msum0, deg2, learnable_skip, W0l, W0r, b0.reshape(1, H))

    x1v = x1_pad.reshape(2 * NPAD, 32)
    (msum1,) = _make_sc_pass(False)(x1v, idxh, z32)

    pooled = pl.pallas_call(
        _layer1_pool_body,
        grid=(16,),
        in_specs=[pl.BlockSpec((BN, H), lambda i: (i, 0)),
                  pl.BlockSpec((BN, H), lambda i: (i, 0)),
                  mspec_a, mspec_b, mspec_a, mspec_b, dspec,
                  pl.BlockSpec((1, 1, BN), lambda i: (i, 0, 0)),
                  _full((2, 2)), _full((2 * H, H)), _full((2 * H, H)),
                  _full((1, H))],
        out_specs=pl.BlockSpec((NG, 3 * H), lambda i: (0, 0)),
        out_shape=jax.ShapeDtypeStruct((NG, 3 * H), F32),
    )(x_pad, x1_pad, msum0, msum0, msum1, msum1, deg2, batch3,
      learnable_skip, W1l, W1r, b1.reshape(1, H))

    out = pl.pallas_call(
        _mlp_body,
        out_shape=jax.ShapeDtypeStruct((NG, H), F32),
    )(pooled, Wp1, bp1.reshape(1, H), Wp2, bp2.reshape(1, H),
      Wp3, bp3.reshape(1, 256), Wp4, bp4.reshape(1, H))

    return out


# speedup vs baseline: 1.3785x; 1.3785x over previous
import functools

import jax
import jax.numpy as jnp
from jax import lax
from jax.experimental import pallas as pl
from jax.experimental.pallas import tpu as pltpu
from jax.experimental.pallas import tpu_sc as plsc

N = 50000
E = 800000
H = 64
NG = 64
NPAD = 50048
BN = NPAD // 16
IDX_W = 128
ROWS_PER_TILE = 400
EPAD = 16 * ROWS_PER_TILE * IDX_W
IDX_ROWS = EPAD // IDX_W
EB = 512
CPB = 5
NBLKE = 20
EPT = NBLKE * CPB * EB
RING = 2
F32 = jnp.float32
BF16 = jnp.bfloat16


def _sc_body(with_deg, *refs):
    if with_deg:
        (xv, idxh, z2, z1, msum_out, deg_out, acc, dega,
         istg0, istg1, rb0, rb1, ones_v, stage,
         sg0, sg1, ss0, ss1, sem_i, sem_d) = refs
    else:
        (xv, idxh, z2, msum_out, acc,
         istg0, istg1, rb0, rb1,
         sg0, sg1, ss0, ss1, sem_i) = refs
    istg = (istg0, istg1)
    rows = (rb0, rb1)
    sem_g = (sg0, sg1)
    sem_s = (ss0, ss1)
    c = lax.axis_index("c")
    s = lax.axis_index("s")
    r0 = s * BN
    pltpu.sync_copy(z2.at[pl.ds(r0, BN)], acc.at[pl.ds(r0, BN)])
    if with_deg:
        pltpu.sync_copy(z1.at[pl.ds(r0, BN)], stage)
        pltpu.sync_copy(stage, dega.at[pl.ds(r0, BN)])
        for j in range(EB // 16):
            ones_v[pl.ds(j * 16, 16)] = jnp.full((16,), 1.0, F32)
    plsc.subcore_barrier()

    ibase = (c * 16 + s) * NBLKE * 2 * CPB
    pltpu.sync_copy(idxh.at[pl.ds(ibase, 2 * CPB)], istg[0])

    def block(b, q):
        ss = istg[q]
        pre = pltpu.async_copy(
            idxh.at[pl.ds(ibase + (b + 1) * 2 * CPB, 2 * CPB)],
            istg[1 - q], sem_i)

        def issue_g(i):
            return pltpu.async_copy(xv.at[ss.at[2 * i]], rows[i % RING],
                                    sem_g[i % RING])

        def issue_s(i):
            return pltpu.async_copy(rows[i % RING], acc.at[ss.at[2 * i + 1]],
                                    sem_s[i % RING], add=True)

        g = {}
        sc = {}
        g[0] = issue_g(0)
        for i in range(CPB):
            g[i].wait()
            sc[i] = issue_s(i)
            if with_deg:
                @pl.when(c == i % 2)
                def _():
                    pltpu.async_copy(ones_v, dega.at[ss.at[2 * i + 1]],
                                     sem_d, add=True)
            if i + 1 < CPB:
                if i - 1 >= 0:
                    sc[i - 1].wait()
                g[i + 1] = issue_g(i + 1)
        if with_deg:
            for i in range(CPB):
                @pl.when(c == i % 2)
                def _():
                    pltpu.make_async_copy(
                        ones_v, dega.at[ss.at[2 * i + 1]], sem_d).wait()
        for t in range(max(0, CPB - RING), CPB):
            sc[t].wait()
        pre.wait()

    def two_blocks(j, carry):
        block(2 * j, 0)
        block(2 * j + 1, 1)
        return carry

    lax.fori_loop(0, NBLKE // 2, two_blocks, 0)
    plsc.subcore_barrier()
    pltpu.sync_copy(acc.at[pl.ds(r0, BN)],
                    msum_out.at[pl.ds(c * NPAD + r0, BN)])
    if with_deg:
        pltpu.sync_copy(dega.at[pl.ds(r0, BN)], stage)
        pltpu.sync_copy(stage, deg_out.at[pl.ds(c * NPAD + r0, BN)])


def _make_sc_pass(with_deg):
    mesh = plsc.VectorSubcoreMesh(core_axis_name="c", subcore_axis_name="s")
    out_type = (jax.ShapeDtypeStruct((2 * NPAD, 32), BF16),)
    if with_deg:
        out_type = out_type + (jax.ShapeDtypeStruct((2 * NPAD,), F32),)
    scratch = [pltpu.VMEM_SHARED((NPAD, 32), BF16)]
    if with_deg:
        scratch.append(pltpu.VMEM_SHARED((NPAD,), F32))
    scratch += [
        pltpu.VMEM((2 * CPB, EB), jnp.int32),
        pltpu.VMEM((2 * CPB, EB), jnp.int32),
        pltpu.VMEM((EB, 32), BF16),
        pltpu.VMEM((EB, 32), BF16),
    ]
    if with_deg:
        scratch.append(pltpu.VMEM((EB,), F32))
        scratch.append(pltpu.VMEM((BN,), F32))
    scratch += [pltpu.SemaphoreType.DMA] * (2 * RING + 1)
    if with_deg:
        scratch.append(pltpu.SemaphoreType.DMA)
    return pl.kernel(
        functools.partial(_sc_body, with_deg),
        out_type=out_type,
        mesh=mesh,
        scratch_types=scratch,
        compiler_params=pltpu.CompilerParams(use_tc_tiling_on_sc=False),
    )


def _pre_body(nf_ref, w_ref, b_ref, o_ref):
    o_ref[...] = (jnp.dot(nf_ref[...], w_ref[...],
                          preferred_element_type=F32) + b_ref[...])


def _layer0_body(x_ref, ma_ref, mb_ref, deg_ref, skip_ref, wl_ref, wr_ref,
                 b_ref, o_ref):
    s00 = jax.nn.sigmoid(skip_ref[0, 0])
    d = jnp.maximum(deg_ref[...], 1.0)
    m = jnp.concatenate([ma_ref[...], mb_ref[...]], axis=1).astype(F32) / d
    h = s00 * (jnp.dot(m, wl_ref[...], preferred_element_type=F32)
               + jnp.dot(x_ref[...], wr_ref[...], preferred_element_type=F32))
    o_ref[...] = jnp.maximum(h + b_ref[...], 0.0)


def _layer1_pool_body(x_ref, x1_ref, m0a_ref, m0b_ref, m1a_ref, m1b_ref,
                      deg_ref, batch_ref, skip_ref, wl_ref, wr_ref, b_ref,
                      pooled_ref):
    i = pl.program_id(0)
    s10 = jax.nn.sigmoid(skip_ref[1, 0])
    s11 = jax.nn.sigmoid(skip_ref[1, 1])
    d = jnp.maximum(deg_ref[...], 1.0)
    m0 = jnp.concatenate([m0a_ref[...], m0b_ref[...]], axis=1).astype(F32) / d
    m1 = jnp.concatenate([m1a_ref[...], m1b_ref[...]], axis=1).astype(F32) / d
    x = x_ref[...]
    x1 = x1_ref[...]
    wl = wl_ref[...]
    wr = wr_ref[...]
    h = (s10 * (jnp.dot(m0, wl[:H], preferred_element_type=F32)
                + jnp.dot(x, wr[:H], preferred_element_type=F32))
         + s11 * (jnp.dot(m1, wl[H:], preferred_element_type=F32)
                  + jnp.dot(x1, wr[H:], preferred_element_type=F32))
         + b_ref[...])
    x2 = jnp.maximum(h, 0.0)
    emb = jnp.concatenate([x, x1, x2], axis=1)
    b = batch_ref[0, 0, :]
    oh = (b[:, None] == lax.broadcasted_iota(jnp.int32, (BN, NG), 1)
          ).astype(F32)
    part = lax.dot_general(oh, emb, (((0,), (0,)), ((), ())),
                           preferred_element_type=F32)

    @pl.when(i == 0)
    def _():
        pooled_ref[...] = part

    @pl.when(i > 0)
    def _():
        pooled_ref[...] += part


def _mlp_body(p_ref, w1_ref, b1_ref, w2_ref, b2_ref, w3_ref, b3_ref,
              w4_ref, b4_ref, o_ref):
    h = jnp.dot(p_ref[...], w1_ref[...], preferred_element_type=F32) + b1_ref[...]
    h = jnp.where(h >= 0, h, 0.1 * h)
    h = jnp.maximum(jnp.dot(h, w2_ref[...], preferred_element_type=F32)
                    + b2_ref[...], 0.0)
    h = jnp.maximum(jnp.dot(h, w3_ref[...], preferred_element_type=F32)
                    + b3_ref[...], 0.0)
    o_ref[...] = jnp.dot(h, w4_ref[...], preferred_element_type=F32) + b4_ref[...]


def _full(shape):
    return pl.BlockSpec(shape, lambda i: tuple(0 for _ in shape))


def kernel(node_feature, edge_index, batch, learnable_skip, W_pre, b_pre,
           W0l, b0, W0r, W1l, b1, W1r,
           Wp1, bp1, Wp2, bp2, Wp3, bp3, Wp4, bp4):
    src = edge_index[0]
    dst = edge_index[1]
    pad = EPAD - E
    srcp = jnp.concatenate([src, jnp.zeros((pad,), jnp.int32)])
    dstp = jnp.concatenate([dst, jnp.full((pad,), N, jnp.int32)])
    dst4 = dstp.reshape(16, NBLKE, CPB, 1, EB)
    idxh = jnp.concatenate(
        [jnp.stack([(2 * srcp + cc).reshape(16, NBLKE, CPB, 1, EB)
                    for cc in range(2)]),
         jnp.stack([dst4, dst4])], axis=4
    ).reshape(2 * 16 * NBLKE * 2 * CPB, EB)
    idxh = jnp.pad(idxh, ((0, 2 * CPB), (0, 0)))
    nf_pad = jnp.pad(node_feature, ((0, NPAD - N), (0, 0)))
    batch3 = jnp.pad(batch, (0, NPAD - N), constant_values=NG
                     ).reshape(16, 1, BN)
    zb = jnp.zeros((NPAD, 32), BF16)
    z1 = jnp.zeros((NPAD,), F32)

    x_pad = pl.pallas_call(
        _pre_body,
        grid=(16,),
        in_specs=[pl.BlockSpec((BN, 5), lambda i: (i, 0)),
                  _full((5, H)), _full((1, H))],
        out_specs=pl.BlockSpec((BN, H), lambda i: (i, 0)),
        out_shape=jax.ShapeDtypeStruct((NPAD, H), F32),
    )(nf_pad, W_pre, b_pre.reshape(1, H))

    xv = x_pad.astype(BF16).reshape(2 * NPAD, 32)
    msum0, degp = _make_sc_pass(True)(xv, idxh, zb, z1)
    deg2 = (degp[:NPAD] + degp[NPAD:]).reshape(NPAD, 1)

    mspec_a = pl.BlockSpec((BN, 32), lambda i: (i, 0))
    mspec_b = pl.BlockSpec((BN, 32), lambda i: (i + 16, 0))
    dspec = pl.BlockSpec((BN, 1), lambda i: (i, 0))
    x1_pad = pl.pallas_call(
        _layer0_body,
        grid=(16,),
        in_specs=[pl.BlockSpec((BN, H), lambda i: (i, 0)),
                  mspec_a, mspec_b, dspec,
                  _full((2, 2)), _full((H, H)), _full((H, H)), _full((1, H))],
        out_specs=pl.BlockSpec((BN, H), lambda i: (i, 0)),
        out_shape=jax.ShapeDtypeStruct((NPAD, H), F32),
    )(x_pad, msum0, msum0, deg2, learnable_skip, W0l, W0r, b0.reshape(1, H))

    x1v = x1_pad.astype(BF16).reshape(2 * NPAD, 32)
    (msum1,) = _make_sc_pass(False)(x1v, idxh, zb)

    pooled = pl.pallas_call(
        _layer1_pool_body,
        grid=(16,),
        in_specs=[pl.BlockSpec((BN, H), lambda i: (i, 0)),
                  pl.BlockSpec((BN, H), lambda i: (i, 0)),
                  mspec_a, mspec_b, mspec_a, mspec_b, dspec,
                  pl.BlockSpec((1, 1, BN), lambda i: (i, 0, 0)),
                  _full((2, 2)), _full((2 * H, H)), _full((2 * H, H)),
                  _full((1, H))],
        out_specs=pl.BlockSpec((NG, 3 * H), lambda i: (0, 0)),
        out_shape=jax.ShapeDtypeStruct((NG, 3 * H), F32),
    )(x_pad, x1_pad, msum0, msum0, msum1, msum1, deg2, batch3,
      learnable_skip, W1l, W1r, b1.reshape(1, H))

    out = pl.pallas_call(
        _mlp_body,
        out_shape=jax.ShapeDtypeStruct((NG, H), F32),
    )(pooled, Wp1, bp1.reshape(1, H), Wp2, bp2.reshape(1, H),
      Wp3, bp3.reshape(1, 256), Wp4, bp4.reshape(1, H))

    return out
